# baseline (speedup 1.0000x reference)
import functools

import jax
import jax.numpy as jnp
from jax import lax
from jax.experimental import pallas as pl
from jax.experimental.pallas import tpu as pltpu
from jax.experimental.pallas import tpu_sc as plsc

N = 10000
E = 320000
D = 128
DH = D // 2
NH = N // 2

NC = 2
NS = 16
CHUNK = 128
BLK = 8
NBLK = 21
CPT = NBLK * BLK
EPT = CPT * CHUNK
E_PAD = EPT * NS
ROWS_PT = 312
ROWS_TAIL = NH - ROWS_PT * NS

_mesh = plsc.VectorSubcoreMesh(core_axis_name="c", subcore_axis_name="s")


@functools.partial(
    pl.kernel,
    out_type=jax.ShapeDtypeStruct((NC, NH, D), jnp.float32),
    mesh=_mesh,
    scratch_types=[
        pltpu.VMEM_SHARED((NH, D), jnp.float32),
        pltpu.VMEM_SHARED((NH, D), jnp.float32),
        pltpu.VMEM((3, BLK, CHUNK), jnp.int32),
        pltpu.VMEM((3, BLK, CHUNK), jnp.int32),
        pltpu.VMEM((3, BLK, CHUNK), jnp.float32),
        pltpu.VMEM((3, BLK, CHUNK), jnp.float32),
        pltpu.VMEM((3, BLK, CHUNK), jnp.float32),
        pltpu.VMEM((2, CHUNK, D), jnp.float32),
        pltpu.SemaphoreType.DMA((3,)),
        pltpu.SemaphoreType.DMA((2,)),
        pltpu.SemaphoreType.DMA((2,)),
        pltpu.SemaphoreType.DMA,
    ],
)
def _sc_msgpass(h_hbm, src_hbm, dst_hbm, sp_hbm, dp_hbm, w_hbm, zero_hbm,
                out_hbm, h_sh, agg_sh, srcS, dstS, spS, dpS, wS, rows, isem,
                gsem, ssem, lsem):
    c = lax.axis_index("c")
    s = lax.axis_index("s")

    rsl = pl.ds(s * ROWS_PT, ROWS_PT)
    st = [
        pltpu.make_async_copy(h_hbm.at[c, rsl], h_sh.at[rsl], lsem),
        pltpu.make_async_copy(zero_hbm.at[pl.ds(0, ROWS_PT)],
                              agg_sh.at[rsl], lsem),
    ]
    for d in st:
        d.start()

    @pl.when(s == NS - 1)
    def _stage_tail():
        tsl = pl.ds(NS * ROWS_PT, ROWS_TAIL)
        pltpu.sync_copy(h_hbm.at[c, tsl], h_sh.at[tsl])
        pltpu.sync_copy(zero_hbm.at[pl.ds(0, ROWS_TAIL)], agg_sh.at[tsl])

    def idx_descs(i, u):
        return [
            pltpu.make_async_copy(src_hbm.at[s, i], srcS.at[u], isem.at[u]),
            pltpu.make_async_copy(dst_hbm.at[s, i], dstS.at[u], isem.at[u]),
            pltpu.make_async_copy(sp_hbm.at[s, i], spS.at[u], isem.at[u]),
            pltpu.make_async_copy(dp_hbm.at[s, i], dpS.at[u], isem.at[u]),
            pltpu.make_async_copy(w_hbm.at[s, i], wS.at[u], isem.at[u]),
        ]

    def idx_start(i, u):
        for d in idx_descs(i, u):
            d.start()

    def idx_wait(i, u):
        for d in idx_descs(i, u):
            d.wait()

    def slot(g):
        i = g // BLK
        return i, g - i * BLK, i % 3, g % 2

    def gather_start(g):
        _, j, u, b = slot(g)
        pltpu.async_copy(h_sh.at[srcS.at[u, j]], rows.at[b], gsem.at[b])

    def gather_wait(g):
        _, j, u, b = slot(g)
        pltpu.make_async_copy(h_sh.at[srcS.at[u, j]], rows.at[b],
                              gsem.at[b]).wait()

    def scatter_start(g):
        _, j, u, b = slot(g)
        pltpu.async_copy(rows.at[b], agg_sh.at[dstS.at[u, j]], ssem.at[b],
                         add=True)

    def scatter_wait(g):
        _, j, u, b = slot(g)
        pltpu.make_async_copy(rows.at[b], agg_sh.at[dstS.at[u, j]],
                              ssem.at[b]).wait()

    idx_start(0, 0)
    idx_start(1, 1)
    idx_wait(0, 0)
    for d in st:
        d.wait()
    plsc.subcore_barrier()
    gather_start(0)

    def chunk_body(g, carry):
        i, j, u, b = slot(g)
        gather_wait(g)

        @pl.when(g >= 1)
        def _drain():
            scatter_wait(g - 1)

        @pl.when((j == 0) & (i + 2 < NBLK))
        def _refill():
            idx_start(i + 2, (u + 2) % 3)

        @pl.when(g + 1 < CPT)
        def _launch():
            @pl.when(j == BLK - 1)
            def _wait_blk():
                idx_wait(i + 1, (u + 1) % 3)

            gather_start(g + 1)

        def row_body(q, carry2):
            wblk = wS[u, j, pl.ds(q * 16, 16)]
            sblk = spS[u, j, pl.ds(q * 16, 16)]
            dblk = dpS[u, j, pl.ds(q * 16, 16)]
            for r2 in range(16):
                rr = q * 16 + r2
                w16 = jnp.broadcast_to(wblk[r2], (16,))
                s16 = jnp.broadcast_to(sblk[r2], (16,))
                w_hi = w16 * jnp.broadcast_to(dblk[r2], (16,))
                w_lo = w16 - w_hi
                for i4 in range(DH // 16):
                    lo = rows[b, rr, pl.ds(i4 * 16, 16)]
                    hi = rows[b, rr, pl.ds(DH + i4 * 16, 16)]
                    m = lo + s16 * (hi - lo)
                    rows[b, rr, pl.ds(i4 * 16, 16)] = m * w_lo
                    rows[b, rr, pl.ds(DH + i4 * 16, 16)] = m * w_hi
            return carry2

        lax.fori_loop(0, CHUNK // 16, row_body, 0)
        scatter_start(g)
        return carry

    lax.fori_loop(0, CPT, chunk_body, 0)
    scatter_wait(CPT - 1)
    plsc.subcore_barrier()

    pltpu.sync_copy(agg_sh.at[rsl], out_hbm.at[c, rsl])

    @pl.when(s == NS - 1)
    def _out_tail():
        tsl = pl.ds(NS * ROWS_PT, ROWS_TAIL)
        pltpu.sync_copy(agg_sh.at[tsl], out_hbm.at[c, tsl])


RB = 2000


def _mm_body(a_ref, w_ref, b_ref, o_ref, *, relu):
    a = jnp.concatenate([a_ref[0], a_ref[1]], axis=1)
    acc = jnp.dot(a, w_ref[...],
                  preferred_element_type=jnp.float32) + b_ref[...]
    if relu:
        acc = jnp.maximum(acc, 0.0)
    o_ref[0] = acc[:, :DH]
    o_ref[1] = acc[:, DH:]


def _mm_res_body(a_ref, w_ref, b_ref, res_ref, o_ref):
    a = jnp.concatenate([a_ref[0], a_ref[1]], axis=1)
    acc = jnp.dot(a, w_ref[...],
                  preferred_element_type=jnp.float32) + b_ref[...]
    acc = jnp.maximum(acc, 0.0)
    o_ref[0] = (res_ref[0] + acc[:, :DH]) * 0.5
    o_ref[1] = (res_ref[1] + acc[:, DH:]) * 0.5


def _mm_final_body(a_ref, w_ref, b_ref, o_ref):
    a = jnp.concatenate([a_ref[0], a_ref[1]], axis=1)
    o_ref[...] = jnp.dot(a, w_ref[...],
                         preferred_element_type=jnp.float32) + b_ref[...]


_HALF_SPEC = pl.BlockSpec((NC, RB, DH), lambda i: (0, i, 0))
_W_SPEC = pl.BlockSpec((D, D), lambda i: (0, 0))
_B_SPEC = pl.BlockSpec((1, D), lambda i: (0, 0))


def _tc_matmul(agg, W, b, relu):
    return pl.pallas_call(
        functools.partial(_mm_body, relu=relu),
        grid=(N // RB,),
        in_specs=[_HALF_SPEC, _W_SPEC, _B_SPEC],
        out_specs=_HALF_SPEC,
        out_shape=jax.ShapeDtypeStruct((NC, N, DH), jnp.float32),
    )(agg.reshape(NC, N, DH), W, b.reshape(1, D)).reshape(NC, NH, D)


def _tc_matmul_res(agg, W, b, res):
    return pl.pallas_call(
        _mm_res_body,
        grid=(N // RB,),
        in_specs=[_HALF_SPEC, _W_SPEC, _B_SPEC, _HALF_SPEC],
        out_specs=_HALF_SPEC,
        out_shape=jax.ShapeDtypeStruct((NC, N, DH), jnp.float32),
    )(agg.reshape(NC, N, DH), W, b.reshape(1, D),
      res.reshape(NC, N, DH)).reshape(NC, NH, D)


def _tc_matmul_final(agg, W, b):
    return pl.pallas_call(
        _mm_final_body,
        grid=(N // RB,),
        in_specs=[_HALF_SPEC, _W_SPEC, _B_SPEC],
        out_specs=pl.BlockSpec((RB, D), lambda i: (i, 0)),
        out_shape=jax.ShapeDtypeStruct((N, D), jnp.float32),
    )(agg.reshape(NC, N, DH), W, b.reshape(1, D))


def kernel(x, edge_index, edge_weight, params):
    src = edge_index[0]
    dst = edge_index[1]
    pad = E_PAD - E
    src_p = jnp.pad(src, (0, pad))
    dst_p = jnp.pad(dst, (0, pad))
    shp = (NS, NBLK, BLK, CHUNK)
    gsrc = (src_p >> 1).reshape(shp)
    gdst = (dst_p >> 1).reshape(shp)
    spf = (src_p & 1).astype(jnp.float32).reshape(shp)
    dpf = (dst_p & 1).astype(jnp.float32).reshape(shp)
    w_p = jnp.pad(edge_weight, (0, pad)).reshape(shp)
    zero = jnp.zeros((ROWS_PT, D), jnp.float32)
    x2 = jnp.stack([x[:, :DH].reshape(NH, D), x[:, DH:].reshape(NH, D)])

    def S(h2):
        return _sc_msgpass(h2, gsrc, gdst, spf, dpf, w_p, zero)

    p_in = params["conv_in"]
    h = _tc_matmul(S(x2), p_in["W"], p_in["b"], relu=True)
    for bp in params["blocks"]:
        h1 = _tc_matmul(S(h), bp["conv1"]["W"], bp["conv1"]["b"], relu=True)
        h = _tc_matmul_res(S(h1), bp["conv2"]["W"], bp["conv2"]["b"], h)
    p_out = params["conv_out"]
    x_out = _tc_matmul_final(S(h), p_out["W"], p_out["b"])
    hu = h.reshape(NC, N, DH)
    x_hidden = jnp.concatenate([hu[0], hu[1]], axis=1)
    return (x_out, x_hidden)


# speedup vs baseline: 2.0103x; 2.0103x over previous
import functools

import jax
import jax.numpy as jnp
from jax import lax
from jax.experimental import pallas as pl
from jax.experimental.pallas import tpu as pltpu
from jax.experimental.pallas import tpu_sc as plsc

N = 10000
E = 320000
D = 128
DH = D // 2
NH = N // 2

NC = 2
NS = 16
CHUNK = 128
BLK = 8
NBLK = 21
CPT = NBLK * BLK
EPT = CPT * CHUNK
E_PAD = EPT * NS
ROWS_PT = 312
ROWS_TAIL = NH - ROWS_PT * NS

_mesh = plsc.VectorSubcoreMesh(core_axis_name="c", subcore_axis_name="s")


@functools.partial(
    pl.kernel,
    out_type=jax.ShapeDtypeStruct((NC, NH, D), jnp.float32),
    mesh=_mesh,
    scratch_types=[
        pltpu.VMEM_SHARED((NH, D), jnp.float32),
        pltpu.VMEM_SHARED((NH, D), jnp.float32),
        pltpu.VMEM((3, BLK, CHUNK), jnp.int32),
        pltpu.VMEM((3, BLK, CHUNK), jnp.int32),
        pltpu.VMEM((3, BLK, CHUNK), jnp.int32),
        pltpu.VMEM((3, BLK, CHUNK), jnp.float32),
        pltpu.VMEM((2, CHUNK, D), jnp.float32),
        pltpu.SemaphoreType.DMA((3,)),
        pltpu.SemaphoreType.DMA((2,)),
        pltpu.SemaphoreType.DMA((2,)),
        pltpu.SemaphoreType.DMA,
    ],
)
def _sc_msgpass(h_hbm, src_hbm, dst_hbm, par_hbm, w_hbm, zero_hbm,
                out_hbm, h_sh, agg_sh, srcS, dstS, parS, wS, rows, isem,
                gsem, ssem, lsem):
    c = lax.axis_index("c")
    s = lax.axis_index("s")

    rsl = pl.ds(s * ROWS_PT, ROWS_PT)
    st = [
        pltpu.make_async_copy(h_hbm.at[c, rsl], h_sh.at[rsl], lsem),
        pltpu.make_async_copy(zero_hbm.at[pl.ds(0, ROWS_PT)],
                              agg_sh.at[rsl], lsem),
    ]
    for d in st:
        d.start()

    @pl.when(s == NS - 1)
    def _stage_tail():
        tsl = pl.ds(NS * ROWS_PT, ROWS_TAIL)
        pltpu.sync_copy(h_hbm.at[c, tsl], h_sh.at[tsl])
        pltpu.sync_copy(zero_hbm.at[pl.ds(0, ROWS_TAIL)], agg_sh.at[tsl])

    def idx_descs(i, u):
        return [
            pltpu.make_async_copy(src_hbm.at[s, i], srcS.at[u], isem.at[u]),
            pltpu.make_async_copy(dst_hbm.at[s, i], dstS.at[u], isem.at[u]),
            pltpu.make_async_copy(par_hbm.at[s, i], parS.at[u], isem.at[u]),
            pltpu.make_async_copy(w_hbm.at[s, i], wS.at[u], isem.at[u]),
        ]

    def idx_start(i, u):
        for d in idx_descs(i, u):
            d.start()

    def idx_wait(i, u):
        for d in idx_descs(i, u):
            d.wait()

    def slot(g):
        i = g // BLK
        return i, g - i * BLK, i % 3, g % 2

    def gather_start(g):
        _, j, u, b = slot(g)
        pltpu.async_copy(h_sh.at[srcS.at[u, j]], rows.at[b], gsem.at[b])

    def gather_wait(g):
        _, j, u, b = slot(g)
        pltpu.make_async_copy(h_sh.at[srcS.at[u, j]], rows.at[b],
                              gsem.at[b]).wait()

    def scatter_start(g):
        _, j, u, b = slot(g)
        pltpu.async_copy(rows.at[b], agg_sh.at[dstS.at[u, j]], ssem.at[b],
                         add=True)

    def scatter_wait(g):
        _, j, u, b = slot(g)
        pltpu.make_async_copy(rows.at[b], agg_sh.at[dstS.at[u, j]],
                              ssem.at[b]).wait()

    idx_start(0, 0)
    idx_start(1, 1)
    idx_wait(0, 0)
    for d in st:
        d.wait()
    plsc.subcore_barrier()
    gather_start(0)

    def chunk_body(g, carry):
        i, j, u, b = slot(g)
        gather_wait(g)

        @pl.when(g >= 1)
        def _drain():
            scatter_wait(g - 1)

        @pl.when((j == 0) & (i + 2 < NBLK))
        def _refill():
            idx_start(i + 2, (u + 2) % 3)

        @pl.when(g + 1 < CPT)
        def _launch():
            @pl.when(j == BLK - 1)
            def _wait_blk():
                idx_wait(i + 1, (u + 1) % 3)

            gather_start(g + 1)

        @plsc.parallel_loop(0, CHUNK // 16, unroll=2)
        def row_body(q):
            wblk = wS[u, j, pl.ds(q * 16, 16)]
            pblk = parS[u, j, pl.ds(q * 16, 16)]
            zz = jnp.zeros((16,), jnp.float32)
            for r2 in range(16):
                rr = q * 16 + r2
                w16 = jnp.broadcast_to(wblk[r2], (16,))
                p = pblk[r2]
                sp = (p & 1) * DH
                dbit = (p >> 1) & 1
                dp = dbit * DH
                od = (1 - dbit) * DH
                for i4 in range(DH // 16):
                    v = rows[b, rr, pl.ds(sp + i4 * 16, 16)]
                    rows[b, rr, pl.ds(dp + i4 * 16, 16)] = v * w16
                for i4 in range(DH // 16):
                    rows[b, rr, pl.ds(od + i4 * 16, 16)] = zz
        scatter_start(g)
        return carry

    lax.fori_loop(0, CPT, chunk_body, 0)
    scatter_wait(CPT - 1)
    plsc.subcore_barrier()

    pltpu.sync_copy(agg_sh.at[rsl], out_hbm.at[c, rsl])

    @pl.when(s == NS - 1)
    def _out_tail():
        tsl = pl.ds(NS * ROWS_PT, ROWS_TAIL)
        pltpu.sync_copy(agg_sh.at[tsl], out_hbm.at[c, tsl])


RB = 2000


def _mm_body(a_ref, w_ref, b_ref, o_ref, *, relu):
    a = jnp.concatenate([a_ref[0], a_ref[1]], axis=1)
    acc = jnp.dot(a, w_ref[...],
                  preferred_element_type=jnp.float32) + b_ref[...]
    if relu:
        acc = jnp.maximum(acc, 0.0)
    o_ref[0] = acc[:, :DH]
    o_ref[1] = acc[:, DH:]


def _mm_res_body(a_ref, w_ref, b_ref, res_ref, o_ref):
    a = jnp.concatenate([a_ref[0], a_ref[1]], axis=1)
    acc = jnp.dot(a, w_ref[...],
                  preferred_element_type=jnp.float32) + b_ref[...]
    acc = jnp.maximum(acc, 0.0)
    o_ref[0] = (res_ref[0] + acc[:, :DH]) * 0.5
    o_ref[1] = (res_ref[1] + acc[:, DH:]) * 0.5


def _mm_final_body(a_ref, w_ref, b_ref, o_ref):
    a = jnp.concatenate([a_ref[0], a_ref[1]], axis=1)
    o_ref[...] = jnp.dot(a, w_ref[...],
                         preferred_element_type=jnp.float32) + b_ref[...]


_HALF_SPEC = pl.BlockSpec((NC, RB, DH), lambda i: (0, i, 0))
_W_SPEC = pl.BlockSpec((D, D), lambda i: (0, 0))
_B_SPEC = pl.BlockSpec((1, D), lambda i: (0, 0))


def _tc_matmul(agg, W, b, relu):
    return pl.pallas_call(
        functools.partial(_mm_body, relu=relu),
        grid=(N // RB,),
        in_specs=[_HALF_SPEC, _W_SPEC, _B_SPEC],
        out_specs=_HALF_SPEC,
        out_shape=jax.ShapeDtypeStruct((NC, N, DH), jnp.float32),
    )(agg.reshape(NC, N, DH), W, b.reshape(1, D)).reshape(NC, NH, D)


def _tc_matmul_res(agg, W, b, res):
    return pl.pallas_call(
        _mm_res_body,
        grid=(N // RB,),
        in_specs=[_HALF_SPEC, _W_SPEC, _B_SPEC, _HALF_SPEC],
        out_specs=_HALF_SPEC,
        out_shape=jax.ShapeDtypeStruct((NC, N, DH), jnp.float32),
    )(agg.reshape(NC, N, DH), W, b.reshape(1, D),
      res.reshape(NC, N, DH)).reshape(NC, NH, D)


def _tc_matmul_final(agg, W, b):
    return pl.pallas_call(
        _mm_final_body,
        grid=(N // RB,),
        in_specs=[_HALF_SPEC, _W_SPEC, _B_SPEC],
        out_specs=pl.BlockSpec((RB, D), lambda i: (i, 0)),
        out_shape=jax.ShapeDtypeStruct((N, D), jnp.float32),
    )(agg.reshape(NC, N, DH), W, b.reshape(1, D))


def kernel(x, edge_index, edge_weight, params):
    src = edge_index[0]
    dst = edge_index[1]
    pad = E_PAD - E
    src_p = jnp.pad(src, (0, pad))
    dst_p = jnp.pad(dst, (0, pad))
    shp = (NS, NBLK, BLK, CHUNK)
    gsrc = (src_p >> 1).reshape(shp)
    gdst = (dst_p >> 1).reshape(shp)
    par = ((src_p & 1) + 2 * (dst_p & 1)).reshape(shp)
    w_p = jnp.pad(edge_weight, (0, pad)).reshape(shp)
    zero = jnp.zeros((ROWS_PT, D), jnp.float32)
    x2 = jnp.stack([x[:, :DH].reshape(NH, D), x[:, DH:].reshape(NH, D)])

    def S(h2):
        return _sc_msgpass(h2, gsrc, gdst, par, w_p, zero)

    p_in = params["conv_in"]
    h = _tc_matmul(S(x2), p_in["W"], p_in["b"], relu=True)
    for bp in params["blocks"]:
        h1 = _tc_matmul(S(h), bp["conv1"]["W"], bp["conv1"]["b"], relu=True)
        h = _tc_matmul_res(S(h1), bp["conv2"]["W"], bp["conv2"]["b"], h)
    p_out = params["conv_out"]
    x_out = _tc_matmul_final(S(h), p_out["W"], p_out["b"])
    hu = h.reshape(NC, N, DH)
    x_hidden = jnp.concatenate([hu[0], hu[1]], axis=1)
    return (x_out, x_hidden)
